# Initial kernel scaffold; baseline (speedup 1.0000x reference)
import functools

import jax
import jax.numpy as jnp
from jax import lax
from jax.experimental import pallas as pl
from jax.experimental.pallas import tpu as pltpu
from jax.experimental.pallas import tpu_sc as plsc

N = 10000
E = 320000
D = 128
N_PAD = 10240
NC = 2
NS = 16
NW = NC * NS
B = 125
CHUNKS = E // B
CPW = CHUNKS // NW
CPT = CHUNKS // NS
RPT = N_PAD // NS
RBLK = 256


def _sc_mesh():
    return plsc.VectorSubcoreMesh(
        core_axis_name="c", subcore_axis_name="s", num_cores=NC, num_subcores=NS
    )


def _deg_kernel_body(e3_hbm, ones_hbm, zeros8_hbm, deg_hbm, idx_v, ones_v, dacc):
    c = lax.axis_index("c")
    s = lax.axis_index("s")
    pltpu.sync_copy(
        zeros8_hbm.at[pl.ds(s * RPT, RPT)], dacc.at[pl.ds(s * RPT, RPT)]
    )
    pltpu.sync_copy(ones_hbm, ones_v)
    pltpu.sync_copy(e3_hbm.at[c, pl.ds(s * CPT, CPT)], idx_v)
    plsc.subcore_barrier()

    def body(j, carry):
        pltpu.sync_copy(ones_v, dacc.at[idx_v.at[j]], add=True)
        return carry

    lax.fori_loop(0, CPT, body, 0)
    plsc.subcore_barrier()
    pltpu.sync_copy(
        dacc.at[pl.ds(s * RPT, RPT)], deg_hbm.at[c, pl.ds(s * RPT, RPT)]
    )


def _make_deg_kernel():
    return functools.partial(
        pl.kernel,
        out_type=jax.ShapeDtypeStruct((NC, N_PAD, 8), jnp.float32),
        mesh=_sc_mesh(),
        scratch_types=[
            pltpu.VMEM((CPT, B), jnp.int32),
            pltpu.VMEM((B, 8), jnp.float32),
            pltpu.VMEM_SHARED((N_PAD, 8), jnp.float32),
        ],
    )(_deg_kernel_body)


def _msg_kernel_body(
    h_hbm, e3_hbm, zeros_hbm, out_hbm, src_v, dst_v, buf0, buf1, acc, sem0, sem1
):
    c = lax.axis_index("c")
    s = lax.axis_index("s")
    w = s * NC + c
    base = w * CPW
    pltpu.sync_copy(e3_hbm.at[0, pl.ds(base, CPW)], src_v)
    pltpu.sync_copy(e3_hbm.at[1, pl.ds(base, CPW)], dst_v)
    pltpu.sync_copy(
        zeros_hbm.at[pl.ds(s * RPT, RPT)], acc.at[pl.ds(s * RPT, RPT)]
    )
    plsc.subcore_barrier()

    pltpu.async_copy(h_hbm.at[src_v.at[0]], buf0, sem0)

    def body(g, carry):
        j = 2 * g
        pltpu.async_copy(h_hbm.at[src_v.at[j + 1]], buf1, sem1)
        pltpu.make_async_copy(h_hbm.at[src_v.at[j]], buf0, sem0).wait()
        pltpu.sync_copy(buf0, acc.at[dst_v.at[j]], add=True)

        @pl.when(j + 2 < CPW)
        def _():
            pltpu.async_copy(h_hbm.at[src_v.at[j + 2]], buf0, sem0)

        pltpu.make_async_copy(h_hbm.at[src_v.at[j + 1]], buf1, sem1).wait()
        pltpu.sync_copy(buf1, acc.at[dst_v.at[j + 1]], add=True)
        return carry

    lax.fori_loop(0, CPW // 2, body, 0)
    plsc.subcore_barrier()
    pltpu.sync_copy(
        acc.at[pl.ds(s * RPT, RPT)], out_hbm.at[c, pl.ds(s * RPT, RPT)]
    )


def _make_msg_kernel():
    return functools.partial(
        pl.kernel,
        out_type=jax.ShapeDtypeStruct((NC, N_PAD, D), jnp.float32),
        mesh=_sc_mesh(),
        scratch_types=[
            pltpu.VMEM((CPW, B), jnp.int32),
            pltpu.VMEM((CPW, B), jnp.int32),
            pltpu.VMEM((B, D), jnp.float32),
            pltpu.VMEM((B, D), jnp.float32),
            pltpu.VMEM_SHARED((N_PAD, D), jnp.float32),
            pltpu.SemaphoreType.DMA,
            pltpu.SemaphoreType.DMA,
        ],
    )(_msg_kernel_body)


def _norms(deg_blk):
    nrm = lax.rsqrt(jnp.maximum(deg_blk, 1.0))
    return nrm[:, 0:1], nrm[:, 1:2]


def _tc1_body(x_ref, deg_ref, w_ref, o_ref):
    ns, _ = _norms(deg_ref[...])
    o_ref[...] = jnp.dot(
        x_ref[...] * ns, w_ref[...], preferred_element_type=jnp.float32
    )


def _tc_mid_body(p_ref, deg_ref, b_ref, w_ref, o_ref):
    ns, nd = _norms(deg_ref[...])
    t = (p_ref[0] + p_ref[1]) * nd + b_ref[...]
    t = jnp.where(t > 0.0, t, jnp.expm1(t))
    o_ref[...] = jnp.dot(t * ns, w_ref[...], preferred_element_type=jnp.float32)


def _tc_out_body(p_ref, deg_ref, b_ref, o_ref):
    _, nd = _norms(deg_ref[...])
    o_ref[...] = (p_ref[0] + p_ref[1]) * nd + b_ref[...]


_GRID = (N_PAD // RBLK,)
_SPEC_ROWS = pl.BlockSpec((RBLK, D), lambda i: (i, 0))
_SPEC_DEG = pl.BlockSpec((RBLK, 2), lambda i: (i, 0))
_SPEC_P = pl.BlockSpec((NC, RBLK, D), lambda i: (0, i, 0))
_SPEC_W = pl.BlockSpec((D, D), lambda i: (0, 0))
_SPEC_B = pl.BlockSpec((1, D), lambda i: (0, 0))
_OUT_ROWS = jax.ShapeDtypeStruct((N_PAD, D), jnp.float32)


def kernel(x, edge_index, W1, b1, W2, b2):
    x_pad = jnp.zeros((N_PAD, D), jnp.float32).at[:N].set(x)
    e3 = edge_index.reshape(2, CHUNKS, B)
    zeros128 = jnp.zeros((N_PAD, D), jnp.float32)
    zeros8 = jnp.zeros((N_PAD, 8), jnp.float32)
    ones8 = jnp.ones((B, 8), jnp.float32)

    deg = _make_deg_kernel()(e3, ones8, zeros8)
    degT = deg[:, :, 0].T

    h1 = pl.pallas_call(
        _tc1_body,
        grid=_GRID,
        in_specs=[_SPEC_ROWS, _SPEC_DEG, _SPEC_W],
        out_specs=_SPEC_ROWS,
        out_shape=_OUT_ROWS,
    )(x_pad, degT, W1)

    msg = _make_msg_kernel()
    p1 = msg(h1, e3, zeros128)

    h2 = pl.pallas_call(
        _tc_mid_body,
        grid=_GRID,
        in_specs=[_SPEC_P, _SPEC_DEG, _SPEC_B, _SPEC_W],
        out_specs=_SPEC_ROWS,
        out_shape=_OUT_ROWS,
    )(p1, degT, b1.reshape(1, D), W2)

    p2 = msg(h2, e3, zeros128)

    logits = pl.pallas_call(
        _tc_out_body,
        grid=_GRID,
        in_specs=[_SPEC_P, _SPEC_DEG, _SPEC_B],
        out_specs=_SPEC_ROWS,
        out_shape=_OUT_ROWS,
    )(p2, degT, b2.reshape(1, D))

    return logits[:N]


# speedup vs baseline: 8.9159x; 8.9159x over previous
import functools

import jax
import jax.numpy as jnp
from jax import lax
from jax.experimental import pallas as pl
from jax.experimental.pallas import tpu as pltpu
from jax.experimental.pallas import tpu_sc as plsc

N = 10000
E = 320000
D = 128
N_PAD = 10240
NC = 2
NS = 16
NW = NC * NS
B = 80
CHUNKS = E // B
CPW = CHUNKS // NW
CPT = CHUNKS // NS
RPT = N_PAD // NS
RBLK = 256


def _sc_mesh():
    return plsc.VectorSubcoreMesh(
        core_axis_name="c", subcore_axis_name="s", num_cores=NC, num_subcores=NS
    )


def _deg_kernel_body(e4_hbm, ones_hbm, zeros8_hbm, deg_hbm, idx_v, ones_v, dacc):
    c = lax.axis_index("c")
    s = lax.axis_index("s")
    pltpu.sync_copy(
        zeros8_hbm.at[pl.ds(s * RPT, RPT)], dacc.at[pl.ds(s * RPT, RPT)]
    )
    pltpu.sync_copy(ones_hbm, ones_v)
    pltpu.sync_copy(e4_hbm.at[c, s], idx_v)
    plsc.subcore_barrier()

    def body(j, carry):
        pltpu.sync_copy(ones_v, dacc.at[idx_v.at[j]], add=True)
        return carry

    lax.fori_loop(0, CPT, body, 0)
    plsc.subcore_barrier()
    pltpu.sync_copy(
        dacc.at[pl.ds(s * RPT, RPT)], deg_hbm.at[c, pl.ds(s * RPT, RPT)]
    )


def _make_deg_kernel():
    return functools.partial(
        pl.kernel,
        out_type=jax.ShapeDtypeStruct((NC, N_PAD, 8), jnp.float32),
        mesh=_sc_mesh(),
        scratch_types=[
            pltpu.VMEM((CPT, B), jnp.int32),
            pltpu.VMEM((B, 8), jnp.float32),
            pltpu.VMEM_SHARED((N_PAD, 8), jnp.float32),
        ],
        compiler_params=pltpu.CompilerParams(use_tc_tiling_on_sc=False),
    )(_deg_kernel_body)


def _msg_kernel_body(
    h_hbm, e4_hbm, zeros_hbm, out_hbm, src_v, dst_v, buf0, buf1, acc, sem0, sem1
):
    c = lax.axis_index("c")
    s = lax.axis_index("s")
    w = s * NC + c
    pltpu.sync_copy(e4_hbm.at[0, w], src_v)
    pltpu.sync_copy(e4_hbm.at[1, w], dst_v)
    pltpu.sync_copy(
        zeros_hbm.at[pl.ds(s * RPT, RPT)], acc.at[pl.ds(s * RPT, RPT)]
    )
    plsc.subcore_barrier()

    pltpu.async_copy(h_hbm.at[src_v.at[0]], buf0, sem0)

    def body(g, carry):
        j = 2 * g
        pltpu.async_copy(h_hbm.at[src_v.at[j + 1]], buf1, sem1)
        pltpu.make_async_copy(h_hbm.at[src_v.at[j]], buf0, sem0).wait()
        pltpu.sync_copy(buf0, acc.at[dst_v.at[j]], add=True)
        pltpu.async_copy(h_hbm.at[src_v.at[j + 2]], buf0, sem0)
        pltpu.make_async_copy(h_hbm.at[src_v.at[j + 1]], buf1, sem1).wait()
        pltpu.sync_copy(buf1, acc.at[dst_v.at[j + 1]], add=True)
        return carry

    lax.fori_loop(0, (CPW - 1) // 2, body, 0)
    pltpu.make_async_copy(h_hbm.at[src_v.at[CPW - 1]], buf0, sem0).wait()
    pltpu.sync_copy(buf0, acc.at[dst_v.at[CPW - 1]], add=True)
    plsc.subcore_barrier()
    pltpu.sync_copy(
        acc.at[pl.ds(s * RPT, RPT)], out_hbm.at[c, pl.ds(s * RPT, RPT)]
    )


def _make_msg_kernel():
    return functools.partial(
        pl.kernel,
        out_type=jax.ShapeDtypeStruct((NC, N_PAD, D), jnp.float32),
        mesh=_sc_mesh(),
        scratch_types=[
            pltpu.VMEM((CPW, B), jnp.int32),
            pltpu.VMEM((CPW, B), jnp.int32),
            pltpu.VMEM((B, D), jnp.float32),
            pltpu.VMEM((B, D), jnp.float32),
            pltpu.VMEM_SHARED((N_PAD, D), jnp.float32),
            pltpu.SemaphoreType.DMA,
            pltpu.SemaphoreType.DMA,
        ],
        compiler_params=pltpu.CompilerParams(use_tc_tiling_on_sc=False),
    )(_msg_kernel_body)


def _norms(deg_blk):
    nrm = lax.rsqrt(jnp.maximum(deg_blk, 1.0))
    return nrm[:, 0:1], nrm[:, 1:2]


def _tc1_body(x_ref, deg_ref, w_ref, o_ref):
    ns, _ = _norms(deg_ref[...])
    o_ref[...] = jnp.dot(
        x_ref[...] * ns, w_ref[...], preferred_element_type=jnp.float32
    )


def _tc_mid_body(p_ref, deg_ref, b_ref, w_ref, o_ref):
    ns, nd = _norms(deg_ref[...])
    t = (p_ref[0] + p_ref[1]) * nd + b_ref[...]
    t = jnp.where(t > 0.0, t, jnp.exp(jnp.minimum(t, 0.0)) - 1.0)
    o_ref[...] = jnp.dot(t * ns, w_ref[...], preferred_element_type=jnp.float32)


def _tc_out_body(p_ref, deg_ref, b_ref, o_ref):
    _, nd = _norms(deg_ref[...])
    o_ref[...] = (p_ref[0] + p_ref[1]) * nd + b_ref[...]


_GRID = (N_PAD // RBLK,)
_SPEC_ROWS = pl.BlockSpec((RBLK, D), lambda i: (i, 0))
_SPEC_DEG = pl.BlockSpec((RBLK, 2), lambda i: (i, 0))
_SPEC_P = pl.BlockSpec((NC, RBLK, D), lambda i: (0, i, 0))
_SPEC_W = pl.BlockSpec((D, D), lambda i: (0, 0))
_SPEC_B = pl.BlockSpec((1, D), lambda i: (0, 0))
_OUT_ROWS = jax.ShapeDtypeStruct((N_PAD, D), jnp.float32)


def kernel(x, edge_index, W1, b1, W2, b2):
    x_pad = jnp.zeros((N_PAD, D), jnp.float32).at[:N].set(x)
    e4m = edge_index.reshape(2, NW, CPW, B)
    e4d = edge_index.reshape(2, NS, CPT, B)
    zeros128 = jnp.zeros((N_PAD, D), jnp.float32)
    zeros8 = jnp.zeros((N_PAD, 8), jnp.float32)
    ones8 = jnp.ones((B, 8), jnp.float32)

    deg = _make_deg_kernel()(e4d, ones8, zeros8)
    degT = deg[:, :, 0].T

    h1 = pl.pallas_call(
        _tc1_body,
        grid=_GRID,
        in_specs=[_SPEC_ROWS, _SPEC_DEG, _SPEC_W],
        out_specs=_SPEC_ROWS,
        out_shape=_OUT_ROWS,
    )(x_pad, degT, W1)

    msg = _make_msg_kernel()
    p1 = msg(h1, e4m, zeros128)

    h2 = pl.pallas_call(
        _tc_mid_body,
        grid=_GRID,
        in_specs=[_SPEC_P, _SPEC_DEG, _SPEC_B, _SPEC_W],
        out_specs=_SPEC_ROWS,
        out_shape=_OUT_ROWS,
    )(p1, degT, b1.reshape(1, D), W2)

    p2 = msg(h2, e4m, zeros128)

    logits = pl.pallas_call(
        _tc_out_body,
        grid=_GRID,
        in_specs=[_SPEC_P, _SPEC_DEG, _SPEC_B],
        out_specs=_SPEC_ROWS,
        out_shape=_OUT_ROWS,
    )(p2, degT, b2.reshape(1, D))

    return logits[:N]


# speedup vs baseline: 11.7815x; 1.3214x over previous
import functools

import jax
import jax.numpy as jnp
from jax import lax
from jax.experimental import pallas as pl
from jax.experimental.pallas import tpu as pltpu
from jax.experimental.pallas import tpu_sc as plsc

N = 10000
E = 320000
D = 128
N_PAD = 10240
NC = 2
NS = 16
NW = NC * NS
B = 80
EPW = E // NW
CPW = EPW // B
EPT = E // NS
BD = 128
CPT = EPT // BD
BD_TAIL = EPT - CPT * BD
RPT = N_PAD // NS
RBLK = 1024


def _sc_mesh():
    return plsc.VectorSubcoreMesh(
        core_axis_name="c", subcore_axis_name="s", num_cores=NC, num_subcores=NS
    )


def _deg_kernel_body(src_hbm, dst_hbm, ones_hbm, zeros8_hbm, deg_hbm,
                     idx_v, ones_v, dacc):
    c = lax.axis_index("c")
    s = lax.axis_index("s")
    pltpu.sync_copy(
        zeros8_hbm.at[pl.ds(s * RPT, RPT)], dacc.at[pl.ds(s * RPT, RPT)]
    )
    pltpu.sync_copy(ones_hbm, ones_v)

    @pl.when(c == 0)
    def _():
        pltpu.sync_copy(src_hbm.at[pl.ds(s * EPT, EPT)], idx_v)

    @pl.when(c == 1)
    def _():
        pltpu.sync_copy(dst_hbm.at[pl.ds(s * EPT, EPT)], idx_v)

    plsc.subcore_barrier()

    def body(j, carry):
        pltpu.sync_copy(ones_v, dacc.at[idx_v.at[pl.ds(j * BD, BD)]], add=True)
        return carry

    lax.fori_loop(0, CPT, body, 0)
    pltpu.sync_copy(
        ones_v.at[pl.ds(0, BD_TAIL)],
        dacc.at[idx_v.at[pl.ds(CPT * BD, BD_TAIL)]],
        add=True,
    )
    plsc.subcore_barrier()
    pltpu.sync_copy(
        dacc.at[pl.ds(s * RPT, RPT)], deg_hbm.at[c, pl.ds(s * RPT, RPT)]
    )


def _make_deg_kernel():
    return functools.partial(
        pl.kernel,
        out_type=jax.ShapeDtypeStruct((NC, N_PAD, 8), jnp.float32),
        mesh=_sc_mesh(),
        scratch_types=[
            pltpu.VMEM((EPT,), jnp.int32),
            pltpu.VMEM((BD, 8), jnp.float32),
            pltpu.VMEM_SHARED((N_PAD, 8), jnp.float32),
        ],
        compiler_params=pltpu.CompilerParams(use_tc_tiling_on_sc=False),
    )(_deg_kernel_body)


def _msg_kernel_body(
    h_hbm, src_hbm, dst_hbm, zeros_hbm, out_hbm,
    src_v, dst_v, buf0, buf1, acc, sem0, sem1
):
    c = lax.axis_index("c")
    s = lax.axis_index("s")
    w = s * NC + c
    pltpu.sync_copy(src_hbm.at[pl.ds(w * EPW, EPW)], src_v)
    pltpu.sync_copy(dst_hbm.at[pl.ds(w * EPW, EPW)], dst_v)
    pltpu.sync_copy(
        zeros_hbm.at[pl.ds(s * RPT, RPT)], acc.at[pl.ds(s * RPT, RPT)]
    )
    plsc.subcore_barrier()

    pltpu.async_copy(h_hbm.at[src_v.at[pl.ds(0, B)]], buf0, sem0)

    def body(g, carry):
        j = 2 * g
        pltpu.async_copy(h_hbm.at[src_v.at[pl.ds((j + 1) * B, B)]], buf1, sem1)
        pltpu.make_async_copy(
            h_hbm.at[src_v.at[pl.ds(j * B, B)]], buf0, sem0
        ).wait()
        pltpu.sync_copy(buf0, acc.at[dst_v.at[pl.ds(j * B, B)]], add=True)
        pltpu.async_copy(h_hbm.at[src_v.at[pl.ds((j + 2) * B, B)]], buf0, sem0)
        pltpu.make_async_copy(
            h_hbm.at[src_v.at[pl.ds((j + 1) * B, B)]], buf1, sem1
        ).wait()
        pltpu.sync_copy(buf1, acc.at[dst_v.at[pl.ds((j + 1) * B, B)]], add=True)
        return carry

    lax.fori_loop(0, (CPW - 1) // 2, body, 0)
    pltpu.make_async_copy(
        h_hbm.at[src_v.at[pl.ds((CPW - 1) * B, B)]], buf0, sem0
    ).wait()
    pltpu.sync_copy(buf0, acc.at[dst_v.at[pl.ds((CPW - 1) * B, B)]], add=True)
    plsc.subcore_barrier()
    pltpu.sync_copy(
        acc.at[pl.ds(s * RPT, RPT)], out_hbm.at[c, pl.ds(s * RPT, RPT)]
    )


def _make_msg_kernel():
    return functools.partial(
        pl.kernel,
        out_type=jax.ShapeDtypeStruct((NC, N_PAD, D), jnp.float32),
        mesh=_sc_mesh(),
        scratch_types=[
            pltpu.VMEM((EPW,), jnp.int32),
            pltpu.VMEM((EPW,), jnp.int32),
            pltpu.VMEM((B, D), jnp.float32),
            pltpu.VMEM((B, D), jnp.float32),
            pltpu.VMEM_SHARED((N_PAD, D), jnp.float32),
            pltpu.SemaphoreType.DMA,
            pltpu.SemaphoreType.DMA,
        ],
        compiler_params=pltpu.CompilerParams(use_tc_tiling_on_sc=False),
    )(_msg_kernel_body)


def _norms(deg_blk):
    nrm = lax.rsqrt(jnp.maximum(deg_blk[:, :, 0:1], 1.0))
    return nrm[0], nrm[1]


def _tc1_body(x_ref, deg_ref, w_ref, o_ref):
    ns, _ = _norms(deg_ref[...])
    o_ref[...] = jnp.dot(
        x_ref[...] * ns, w_ref[...], preferred_element_type=jnp.float32
    )


def _tc_mid_body(p_ref, deg_ref, b_ref, w_ref, o_ref):
    ns, nd = _norms(deg_ref[...])
    t = (p_ref[0] + p_ref[1]) * nd + b_ref[...]
    t = jnp.where(t > 0.0, t, jnp.exp(jnp.minimum(t, 0.0)) - 1.0)
    o_ref[...] = jnp.dot(t * ns, w_ref[...], preferred_element_type=jnp.float32)


def _tc_out_body(p_ref, deg_ref, b_ref, o_ref):
    _, nd = _norms(deg_ref[...])
    o_ref[...] = (p_ref[0] + p_ref[1]) * nd + b_ref[...]


_GRID = (N_PAD // RBLK,)
_SPEC_ROWS = pl.BlockSpec((RBLK, D), lambda i: (i, 0))
_SPEC_DEG = pl.BlockSpec((NC, RBLK, 8), lambda i: (0, i, 0))
_SPEC_P = pl.BlockSpec((NC, RBLK, D), lambda i: (0, i, 0))
_SPEC_W = pl.BlockSpec((D, D), lambda i: (0, 0))
_SPEC_B = pl.BlockSpec((1, D), lambda i: (0, 0))
_OUT_ROWS = jax.ShapeDtypeStruct((N_PAD, D), jnp.float32)


def kernel(x, edge_index, W1, b1, W2, b2):
    src = edge_index[0]
    dst = edge_index[1]
    zeros128 = jnp.zeros((N_PAD, D), jnp.float32)
    zeros8 = jnp.zeros((N_PAD, 8), jnp.float32)
    ones8 = jnp.ones((BD, 8), jnp.float32)

    deg = _make_deg_kernel()(src, dst, ones8, zeros8)

    h1 = pl.pallas_call(
        _tc1_body,
        grid=_GRID,
        in_specs=[_SPEC_ROWS, _SPEC_DEG, _SPEC_W],
        out_specs=_SPEC_ROWS,
        out_shape=_OUT_ROWS,
    )(x, deg, W1)

    msg = _make_msg_kernel()
    p1 = msg(h1, src, dst, zeros128)

    h2 = pl.pallas_call(
        _tc_mid_body,
        grid=_GRID,
        in_specs=[_SPEC_P, _SPEC_DEG, _SPEC_B, _SPEC_W],
        out_specs=_SPEC_ROWS,
        out_shape=_OUT_ROWS,
    )(p1, deg, b1.reshape(1, D), W2)

    p2 = msg(h2, src, dst, zeros128)

    logits = pl.pallas_call(
        _tc_out_body,
        grid=_GRID,
        in_specs=[_SPEC_P, _SPEC_DEG, _SPEC_B],
        out_specs=_SPEC_ROWS,
        out_shape=jax.ShapeDtypeStruct((N, D), jnp.float32),
    )(p2, deg, b2.reshape(1, D))

    return logits
